# baseline (speedup 1.0000x reference)
import jax
import jax.numpy as jnp
from jax import lax
from jax.experimental import pallas as pl
from jax.experimental.pallas import tpu as pltpu
from jax.experimental.pallas import tpu_sc as plsc

B = 16384
N = 200
N_BINS = 22
N_DIM = 4
OUT_COLS = N_BINS * N_DIM
LANES = 16
HIST_WORDS = LANES * N_BINS
T_STRIDE = 89
NUM_CORES = 2
NUM_SUBCORES = 16
NW = NUM_CORES * NUM_SUBCORES
ROWS_PER_W = B // NW
GROUP = LANES
CHUNK = 64
GROUPS_PER_CHUNK = CHUNK // GROUP
CHUNKS = ROWS_PER_W // CHUNK
LUT_SIZE = 256
OUT_STAGE = CHUNK * OUT_COLS
OUT_VREGS = 6


def _sc_hist_body(cos_hbm, lut2_hbm, out_hbm, in_v0, in_v1, out_v0, out_v1,
                  hist_v, lut2_v, si0, si1, so0, so1):
    wid = lax.axis_index("s") * NUM_CORES + lax.axis_index("c")
    row0 = wid * ROWS_PER_W

    pltpu.sync_copy(lut2_hbm, lut2_v)

    lanes = lax.iota(jnp.int32, LANES)
    rot_base = lanes * (N + 1)
    row_end = lanes * N + N
    cbin0 = lanes + (10 << 4)
    cbin1 = lanes + (11 << 4)
    lane88 = lanes * OUT_COLS
    ones = jnp.full((LANES,), 1.0, dtype=jnp.float32)
    zeros = jnp.zeros((LANES,), dtype=jnp.float32)

    in_bufs = (in_v0, in_v1)
    out_bufs = (out_v0, out_v1)
    sin = (si0, si1)
    sout = (so0, so1)

    def in_slice(cc):
        return cos_hbm.at[pl.ds(row0 + cc * CHUNK, CHUNK)]

    def out_slice(cc):
        return out_hbm.at[pl.ds(row0 + cc * CHUNK, CHUNK)]

    pltpu.async_copy(in_slice(0), in_v0, si0)
    pltpu.async_copy(in_slice(1), in_v1, si1)

    @pl.loop(0, CHUNKS, step=2)
    def _chunk_loop(c):
        for b in range(2):
            cc = c + b
            in_v = in_bufs[b]
            out_v = out_bufs[b]
            pltpu.make_async_copy(in_slice(cc), in_v, sin[b]).wait()

            @pl.when(cc >= 2)
            def _wait_out():
                pltpu.make_async_copy(out_v, out_slice(cc - 2),
                                      sout[b]).wait()

            @pl.loop(0, GROUPS_PER_CHUNK)
            def _group_loop(g):
                for i in range(HIST_WORDS // LANES):
                    hist_v[pl.ds(i * LANES, LANES)] = zeros

                grow = lanes + g * GROUP

                def bin_scatter(vals):
                    y = vals * jnp.float32(10.0)
                    t = y.astype(jnp.int32)
                    tf = t.astype(jnp.float32)
                    idx = (t << 4) + jnp.where(tf < y, cbin1, cbin0)
                    plsc.addupdate_scatter(hist_v, [idx], ones)

                @plsc.parallel_loop(0, 128, unroll=8)
                def _val_loop(n):
                    col = jnp.full((LANES,), n, dtype=jnp.int32)
                    bin_scatter(plsc.load_gather(in_v, [grow, col]))

                @plsc.parallel_loop(128, N, unroll=8)
                def _val_hi(n):
                    col = jnp.full((LANES,), n, dtype=jnp.int32)
                    bin_scatter(plsc.load_gather(in_v, [grow, col]))

                @plsc.parallel_loop(0, N_BINS, unroll=2)
                def _out_loop(j):
                    cnt = hist_v[pl.ds(j * LANES, LANES)]
                    ci = cnt.astype(jnp.int32)
                    for d in range(N_DIM):
                        lg = plsc.load_gather(
                            lut2_v, [ci + (j * (N_DIM * LUT_SIZE) +
                                           d * LUT_SIZE)])
                        colv = jnp.full((LANES,), 4 * j + d, dtype=jnp.int32)
                        plsc.store_scatter(out_v, [grow, colv], lg)

            @pl.when(cc + 2 < CHUNKS)
            def _prefetch_in():
                pltpu.async_copy(in_slice(cc + 2), in_v, sin[b])

            pltpu.async_copy(out_v, out_slice(cc), sout[b])

    for b in range(2):
        pltpu.make_async_copy(out_bufs[b], out_slice(CHUNKS - 2 + b),
                              sout[b]).wait()


_sc_hist_kernel = None


def _get_sc_kernel():
    global _sc_hist_kernel
    if _sc_hist_kernel is None:
        mesh = plsc.VectorSubcoreMesh(
            core_axis_name="c",
            subcore_axis_name="s",
            num_cores=NUM_CORES,
            num_subcores=NUM_SUBCORES,
        )
        _sc_hist_kernel = pl.kernel(
            _sc_hist_body,
            out_type=jax.ShapeDtypeStruct((B, OUT_COLS), jnp.float32),
            mesh=mesh,
            scratch_types=[
                pltpu.VMEM((CHUNK, N), jnp.float32),
                pltpu.VMEM((CHUNK, N), jnp.float32),
                pltpu.VMEM((CHUNK, OUT_COLS), jnp.float32),
                pltpu.VMEM((CHUNK, OUT_COLS), jnp.float32),
                pltpu.VMEM((HIST_WORDS,), jnp.float32),
                pltpu.VMEM((OUT_COLS * LUT_SIZE,), jnp.float32),
                pltpu.SemaphoreType.DMA,
                pltpu.SemaphoreType.DMA,
                pltpu.SemaphoreType.DMA,
                pltpu.SemaphoreType.DMA,
            ],
            compiler_params=pltpu.CompilerParams(needs_layout_passes=False),
        )
    return _sc_hist_kernel


def kernel(cosine, emb):
    lut = jnp.log(jnp.arange(LUT_SIZE, dtype=jnp.float32) + 1.0)
    lut2 = (emb.reshape(OUT_COLS, 1) * lut.reshape(1, LUT_SIZE)).reshape(-1)
    return _get_sc_kernel()(cosine, lut2)


# speedup vs baseline: 1.3704x; 1.3704x over previous
import jax
import jax.numpy as jnp
from jax import lax
from jax.experimental import pallas as pl
from jax.experimental.pallas import tpu as pltpu
from jax.experimental.pallas import tpu_sc as plsc

B = 16384
N = 200
N_BINS = 22
N_DIM = 4
OUT_COLS = N_BINS * N_DIM
LANES = 16
HIST_WORDS = LANES * N_BINS
T_STRIDE = 89
NUM_CORES = 2
NUM_SUBCORES = 16
NW = NUM_CORES * NUM_SUBCORES
ROWS_PER_W = B // NW
GROUP = LANES
CHUNK = 64
GROUPS_PER_CHUNK = CHUNK // GROUP
CHUNKS = ROWS_PER_W // CHUNK
LUT_SIZE = 256
OUT_STAGE = CHUNK * OUT_COLS
OUT_VREGS = 6


def _sc_hist_body(cos_hbm, lut2_hbm, out_hbm, in_v0, in_v1, out_v0, out_v1,
                  hist_v, lut2_v, si0, si1, so0, so1):
    wid = lax.axis_index("s") * NUM_CORES + lax.axis_index("c")
    row0 = wid * ROWS_PER_W

    pltpu.sync_copy(lut2_hbm, lut2_v)

    lanes = lax.iota(jnp.int32, LANES)
    rot_base = lanes * (N + 1)
    row_end = lanes * N + N
    cbin0 = lanes + (10 << 4)
    cbin1 = lanes + (11 << 4)
    lanes128 = lanes - 128
    ones = jnp.full((LANES,), 1.0, dtype=jnp.float32)
    zeros = jnp.zeros((LANES,), dtype=jnp.float32)

    in_bufs = (in_v0, in_v1)
    out_bufs = (out_v0, out_v1)
    sin = (si0, si1)
    sout = (so0, so1)

    def in_slice(cc):
        return cos_hbm.at[pl.ds(row0 + cc * CHUNK, CHUNK)]

    def out_slice(cc):
        return out_hbm.at[pl.ds(row0 + cc * CHUNK, CHUNK)]

    pltpu.async_copy(in_slice(0), in_v0, si0)
    pltpu.async_copy(in_slice(1), in_v1, si1)

    @pl.loop(0, CHUNKS, step=2)
    def _chunk_loop(c):
        for b in range(2):
            cc = c + b
            in_v = in_bufs[b]
            out_v = out_bufs[b]
            pltpu.make_async_copy(in_slice(cc), in_v, sin[b]).wait()

            @pl.when(cc >= 2)
            def _wait_out():
                pltpu.make_async_copy(out_v, out_slice(cc - 2),
                                      sout[b]).wait()

            @pl.loop(0, GROUPS_PER_CHUNK)
            def _group_loop(g):
                for i in range(HIST_WORDS // LANES):
                    hist_v[pl.ds(i * LANES, LANES)] = zeros

                grow = lanes + g * GROUP

                def bin_scatter(vals):
                    y = vals * jnp.float32(10.0)
                    t = y.astype(jnp.int32)
                    tf = t.astype(jnp.float32)
                    idx = (t << 4) + jnp.where(tf < y, cbin1, cbin0)
                    plsc.addupdate_scatter(hist_v, [idx], ones)

                @plsc.parallel_loop(0, 128 - (LANES - 1), unroll=8)
                def _val_lo(n):
                    col = (lanes + n) & 127
                    bin_scatter(plsc.load_gather(in_v, [grow, col]))

                @plsc.parallel_loop(128 - (LANES - 1), 128, unroll=5)
                def _val_mid(n):
                    col = lanes + n
                    bin_scatter(plsc.load_gather(in_v, [grow, col]))

                @plsc.parallel_loop(128, N - (LANES - 1), unroll=8)
                def _val_hi(n):
                    col = ((lanes128 + n) & 127) + 128
                    bin_scatter(plsc.load_gather(in_v, [grow, col]))

                @plsc.parallel_loop(N - (LANES - 1), N, unroll=5)
                def _val_tail(n):
                    col = lanes + n
                    col = jnp.where(col >= N, col - N, col)
                    bin_scatter(plsc.load_gather(in_v, [grow, col]))

                @plsc.parallel_loop(0, N_BINS, unroll=2)
                def _out_loop(j):
                    cnt = hist_v[pl.ds(j * LANES, LANES)]
                    ci = cnt.astype(jnp.int32)
                    for d in range(N_DIM):
                        lg = plsc.load_gather(
                            lut2_v, [ci + (j * (N_DIM * LUT_SIZE) +
                                           d * LUT_SIZE)])
                        colv = jnp.full((LANES,), 4 * j + d, dtype=jnp.int32)
                        plsc.store_scatter(out_v, [grow, colv], lg)

            @pl.when(cc + 2 < CHUNKS)
            def _prefetch_in():
                pltpu.async_copy(in_slice(cc + 2), in_v, sin[b])

            pltpu.async_copy(out_v, out_slice(cc), sout[b])

    for b in range(2):
        pltpu.make_async_copy(out_bufs[b], out_slice(CHUNKS - 2 + b),
                              sout[b]).wait()


_sc_hist_kernel = None


def _get_sc_kernel():
    global _sc_hist_kernel
    if _sc_hist_kernel is None:
        mesh = plsc.VectorSubcoreMesh(
            core_axis_name="c",
            subcore_axis_name="s",
            num_cores=NUM_CORES,
            num_subcores=NUM_SUBCORES,
        )
        _sc_hist_kernel = pl.kernel(
            _sc_hist_body,
            out_type=jax.ShapeDtypeStruct((B, OUT_COLS), jnp.float32),
            mesh=mesh,
            scratch_types=[
                pltpu.VMEM((CHUNK, N), jnp.float32),
                pltpu.VMEM((CHUNK, N), jnp.float32),
                pltpu.VMEM((CHUNK, OUT_COLS), jnp.float32),
                pltpu.VMEM((CHUNK, OUT_COLS), jnp.float32),
                pltpu.VMEM((HIST_WORDS,), jnp.float32),
                pltpu.VMEM((OUT_COLS * LUT_SIZE,), jnp.float32),
                pltpu.SemaphoreType.DMA,
                pltpu.SemaphoreType.DMA,
                pltpu.SemaphoreType.DMA,
                pltpu.SemaphoreType.DMA,
            ],
            compiler_params=pltpu.CompilerParams(needs_layout_passes=False),
        )
    return _sc_hist_kernel


def kernel(cosine, emb):
    lut = jnp.log(jnp.arange(LUT_SIZE, dtype=jnp.float32) + 1.0)
    lut2 = (emb.reshape(OUT_COLS, 1) * lut.reshape(1, LUT_SIZE)).reshape(-1)
    return _get_sc_kernel()(cosine, lut2)


# speedup vs baseline: 1.3719x; 1.0011x over previous
import jax
import jax.numpy as jnp
from jax import lax
from jax.experimental import pallas as pl
from jax.experimental.pallas import tpu as pltpu
from jax.experimental.pallas import tpu_sc as plsc

B = 16384
N = 200
N_BINS = 22
N_DIM = 4
OUT_COLS = N_BINS * N_DIM
LANES = 16
HIST_WORDS = LANES * N_BINS
NUM_CORES = 2
NUM_SUBCORES = 16
NW = NUM_CORES * NUM_SUBCORES
ROWS_PER_W = B // NW
GROUP = LANES
CHUNK = 64
GROUPS_PER_CHUNK = CHUNK // GROUP
CHUNKS = ROWS_PER_W // CHUNK
LUT_SIZE = 256


def _sc_hist_body(cos_hbm, lut2_hbm, out_hbm, in_v0, in_v1, out_v0, out_v1,
                  hist_v, lut2_v, si0, si1, so0, so1):
    wid = lax.axis_index("s") * NUM_CORES + lax.axis_index("c")
    row0 = wid * ROWS_PER_W

    pltpu.sync_copy(lut2_hbm, lut2_v)

    lanes = lax.iota(jnp.int32, LANES)
    cbin0 = lanes + (10 << 4)
    cbin1 = lanes + (11 << 4)
    lanes128 = lanes - 128
    ones = jnp.full((LANES,), 1.0, dtype=jnp.float32)
    zeros = jnp.zeros((LANES,), dtype=jnp.float32)

    in_bufs = (in_v0, in_v1)
    out_bufs = (out_v0, out_v1)
    sin = (si0, si1)
    sout = (so0, so1)

    def in_slice(cc):
        return cos_hbm.at[pl.ds(row0 + cc * CHUNK, CHUNK)]

    def out_slice(cc):
        return out_hbm.at[pl.ds(row0 + cc * CHUNK, CHUNK)]

    pltpu.async_copy(in_slice(0), in_v0, si0)
    pltpu.async_copy(in_slice(1), in_v1, si1)

    @pl.loop(0, CHUNKS, step=2)
    def _chunk_loop(c):
        for b in range(2):
            cc = c + b
            in_v = in_bufs[b]
            out_v = out_bufs[b]
            pltpu.make_async_copy(in_slice(cc), in_v, sin[b]).wait()

            @pl.when(cc >= 2)
            def _wait_out():
                pltpu.make_async_copy(out_v, out_slice(cc - 2),
                                      sout[b]).wait()

            @pl.loop(0, GROUPS_PER_CHUNK)
            def _group_loop(g):
                for i in range(HIST_WORDS // LANES):
                    hist_v[pl.ds(i * LANES, LANES)] = zeros

                grow = lanes + g * GROUP

                def bin_scatter(vals):
                    y = vals * jnp.float32(10.0)
                    t = y.astype(jnp.int32)
                    tf = t.astype(jnp.float32)
                    idx = (t << 4) + jnp.where(tf < y, cbin1, cbin0)
                    plsc.addupdate_scatter(hist_v, [idx], ones)

                @plsc.parallel_loop(0, 128 - (LANES - 1), unroll=8)
                def _val_lo(n):
                    col = (lanes + n) & 127
                    bin_scatter(plsc.load_gather(in_v, [grow, col]))

                @plsc.parallel_loop(128 - (LANES - 1), 128, unroll=5)
                def _val_mid(n):
                    col = lanes + n
                    bin_scatter(plsc.load_gather(in_v, [grow, col]))

                @plsc.parallel_loop(128, N - (LANES - 1), unroll=8)
                def _val_hi(n):
                    col = ((lanes128 + n) & 127) + 128
                    bin_scatter(plsc.load_gather(in_v, [grow, col]))

                @plsc.parallel_loop(N - (LANES - 1), N, unroll=5)
                def _val_tail(n):
                    col = lanes + n
                    col = jnp.where(col >= N, col - N, col)
                    bin_scatter(plsc.load_gather(in_v, [grow, col]))

                @plsc.parallel_loop(0, N_BINS, unroll=2)
                def _out_loop(j):
                    cnt = hist_v[pl.ds(j * LANES, LANES)]
                    ci = cnt.astype(jnp.int32)
                    for d in range(N_DIM):
                        lg = plsc.load_gather(
                            lut2_v, [ci + (j * (N_DIM * LUT_SIZE) +
                                           d * LUT_SIZE)])
                        colv = jnp.full((LANES,), 4 * j + d, dtype=jnp.int32)
                        plsc.store_scatter(out_v, [grow, colv], lg)

            @pl.when(cc + 2 < CHUNKS)
            def _prefetch_in():
                pltpu.async_copy(in_slice(cc + 2), in_v, sin[b])

            pltpu.async_copy(out_v, out_slice(cc), sout[b])

    for b in range(2):
        pltpu.make_async_copy(out_bufs[b], out_slice(CHUNKS - 2 + b),
                              sout[b]).wait()


_sc_hist_kernel = None


def _get_sc_kernel():
    global _sc_hist_kernel
    if _sc_hist_kernel is None:
        mesh = plsc.VectorSubcoreMesh(
            core_axis_name="c",
            subcore_axis_name="s",
            num_cores=NUM_CORES,
            num_subcores=NUM_SUBCORES,
        )
        _sc_hist_kernel = pl.kernel(
            _sc_hist_body,
            out_type=jax.ShapeDtypeStruct((B, OUT_COLS), jnp.float32),
            mesh=mesh,
            scratch_types=[
                pltpu.VMEM((CHUNK, N), jnp.float32),
                pltpu.VMEM((CHUNK, N), jnp.float32),
                pltpu.VMEM((CHUNK, OUT_COLS), jnp.float32),
                pltpu.VMEM((CHUNK, OUT_COLS), jnp.float32),
                pltpu.VMEM((HIST_WORDS,), jnp.float32),
                pltpu.VMEM((OUT_COLS * LUT_SIZE,), jnp.float32),
                pltpu.SemaphoreType.DMA,
                pltpu.SemaphoreType.DMA,
                pltpu.SemaphoreType.DMA,
                pltpu.SemaphoreType.DMA,
            ],
            compiler_params=pltpu.CompilerParams(needs_layout_passes=False),
        )
    return _sc_hist_kernel


def kernel(cosine, emb):
    lut = jnp.log(jnp.arange(LUT_SIZE, dtype=jnp.float32) + 1.0)
    lut2 = (emb.reshape(OUT_COLS, 1) * lut.reshape(1, LUT_SIZE)).reshape(-1)
    return _get_sc_kernel()(cosine, lut2)
